# Initial kernel scaffold; baseline (speedup 1.0000x reference)
import jax
import jax.numpy as jnp
from jax.experimental import pallas as pl


def kernel(x):
    raise NotImplementedError("write your pallas kernel here")


# speedup vs baseline: 1.2928x; 1.2928x over previous
import functools

import jax
import jax.numpy as jnp
import numpy as np
from jax import lax
from jax.experimental import pallas as pl
from jax.experimental.pallas import tpu as pltpu
from jax.experimental.pallas import tpu_sc as plsc

_COUNTS = [15, 8, 8]
B, C, N = 32, 256, 64
ROWS = B * C
NW = 32
RPW = ROWS // NW
T = 8
TILE = N * N
NCHUNK = RPW // T
NPAIR = NCHUNK // 2


def _static_mask() -> np.ndarray:
    mask = np.eye(N, dtype=bool)
    stride, offset = 1, 0
    for _, count in enumerate(_COUNTS):
        for _ in range(count):
            ii = np.arange(0, N - offset, stride)
            mask[ii, ii + offset] = True
            offset += stride
        offset += stride
        stride *= 2
    return mask


_MASK2D = _static_mask()

_mesh = plsc.VectorSubcoreMesh(
    core_axis_name="c", subcore_axis_name="s", num_cores=2, num_subcores=16
)


@functools.partial(
    pl.kernel,
    out_type=jax.ShapeDtypeStruct((ROWS * TILE,), jnp.float32),
    mesh=_mesh,
    compiler_params=pltpu.CompilerParams(
        needs_layout_passes=False, use_tc_tiling_on_sc=False
    ),
    scratch_types=[
        pltpu.VMEM((RPW * N + 16,), jnp.float32),
        pltpu.VMEM((T * TILE,), jnp.float32),
        pltpu.VMEM((T * TILE,), jnp.float32),
        pltpu.VMEM((128,), jnp.float32),
        pltpu.VMEM((64,), jnp.float32),
        pltpu.VMEM((32,), jnp.float32),
        pltpu.SemaphoreType.DMA,
        pltpu.SemaphoreType.DMA,
    ],
)
def _sc_kernel(x_hbm, out_hbm, in_buf, ob0, ob1, curbuf, zbuf, wbuf, sem0, sem1):
    wid = lax.axis_index("s") * 2 + lax.axis_index("c")
    row0 = wid * RPW

    pltpu.sync_copy(
        x_hbm.at[pl.ds(row0 * N, RPW * N)], in_buf.at[pl.ds(0, RPW * N)]
    )

    iota = lax.iota(jnp.int32, 16)
    zeros16 = jnp.zeros((16,), jnp.float32)
    masks = {c: iota < c for c in range(1, 16)}
    ibase = [iota * 65 + 1040 * k for k in range(4)]
    i130 = iota * 130
    i260 = iota * 260
    g2i = iota * 2

    def zero_body(j, _):
        for u in range(4):
            ob0[pl.ds(j * 64 + u * 16, 16)] = zeros16
            ob1[pl.ds(j * 64 + u * 16, 16)] = zeros16
        return 0

    lax.fori_loop(0, T * TILE // 64, zero_body, 0)

    obs = (ob0, ob1)
    sems = (sem0, sem1)

    def compute_chunk(g, ob):
        def row_body(t, _):
            base_in = (g * T + t) * N
            tb = t * TILE
            c = [in_buf[pl.ds(base_in + 16 * k, 16)] for k in range(4)]
            for o in range(15):
                if o:
                    c = [
                        jnp.maximum(c[k], in_buf[pl.ds(base_in + 16 * k + o, 16)])
                        for k in range(4)
                    ]
                for k in range(4):
                    idx = ibase[k] + (tb + o)
                    if k == 3 and o:
                        plsc.store_scatter(ob, [idx], c[k], mask=masks[16 - o])
                    else:
                        plsc.store_scatter(ob, [idx], c[k])
            for k in range(4):
                curbuf[pl.ds(16 * k, 16)] = c[k]
            z0 = jnp.maximum(
                jnp.maximum(
                    plsc.load_gather(curbuf, [g2i]),
                    plsc.load_gather(curbuf, [g2i + 1]),
                ),
                plsc.load_gather(curbuf, [g2i + 2]),
            )
            z1 = jnp.maximum(
                jnp.maximum(
                    plsc.load_gather(curbuf, [g2i + 32]),
                    plsc.load_gather(curbuf, [g2i + 33]),
                ),
                plsc.load_gather(curbuf, [g2i + 34]),
            )
            for o in range(8):
                plsc.store_scatter(ob, [i130 + (tb + 16 + 2 * o)], z0)
                plsc.store_scatter(
                    ob, [i130 + (tb + 2096 + 2 * o)], z1, mask=masks[8 - o]
                )
                zbuf[pl.ds(0, 16)] = z0
                zbuf[pl.ds(16, 16)] = z1
                if o < 7:
                    z0 = jnp.maximum(z0, zbuf[pl.ds(1, 16)])
                    z1 = jnp.maximum(z1, zbuf[pl.ds(17, 16)])
            w = jnp.maximum(
                jnp.maximum(
                    plsc.load_gather(zbuf, [g2i]),
                    plsc.load_gather(zbuf, [g2i + 1]),
                ),
                plsc.load_gather(zbuf, [g2i + 2]),
            )
            for o in range(8):
                plsc.store_scatter(
                    ob, [i260 + (tb + 34 + 4 * o)], w, mask=masks[8 - o]
                )
                wbuf[pl.ds(0, 16)] = w
                if o < 7:
                    w = jnp.maximum(w, wbuf[pl.ds(1, 16)])
            return 0

        lax.fori_loop(0, T, row_body, 0)

    def start(g, ob, sem):
        off = (row0 + g * T) * TILE
        pltpu.async_copy(ob, out_hbm.at[pl.ds(off, T * TILE)], sem)

    def wait(g, ob, sem):
        off = (row0 + g * T) * TILE
        pltpu.make_async_copy(ob, out_hbm.at[pl.ds(off, T * TILE)], sem).wait()

    for b in range(2):
        compute_chunk(b, obs[b])
        start(b, obs[b], sems[b])

    def pair_body(p, _):
        for b in range(2):
            g = 2 * p + b
            wait(g - 2, obs[b], sems[b])
            compute_chunk(g, obs[b])
            start(g, obs[b], sems[b])
        return 0

    lax.fori_loop(1, NPAIR, pair_body, 0)

    for b in range(2):
        wait(NCHUNK - 2 + b, obs[b], sems[b])


@jax.jit
def kernel(x):
    out = _sc_kernel(x.reshape(ROWS * N))
    x2d = out.reshape(B, C, N, N)
    return x2d, jnp.asarray(_MASK2D)


# speedup vs baseline: 2.2756x; 1.7602x over previous
import functools

import jax
import jax.numpy as jnp
import numpy as np
from jax import lax
from jax.experimental import pallas as pl
from jax.experimental.pallas import tpu as pltpu
from jax.experimental.pallas import tpu_sc as plsc

_COUNTS = [15, 8, 8]
B, C, N = 32, 256, 64
NW = 32
ROW = N * C
BLK = N * C
VPR = C // 16

_L0 = set(range(15))
_L1 = set(range(16, 31, 2))
_L2 = set(range(34, 63, 4))
_O_CLASS = [_L0 | _L1 | _L2, _L0, _L0 | _L1, _L0]
_CLASS_DEPTH = [62, 14, 30, 14]


def _static_mask() -> np.ndarray:
    mask = np.eye(N, dtype=bool)
    stride, offset = 1, 0
    for _, count in enumerate(_COUNTS):
        for _ in range(count):
            ii = np.arange(0, N - offset, stride)
            mask[ii, ii + offset] = True
            offset += stride
        offset += stride
        stride *= 2
    return mask


_MASK2D = _static_mask()

_mesh = plsc.VectorSubcoreMesh(
    core_axis_name="c", subcore_axis_name="s", num_cores=2, num_subcores=16
)


@functools.partial(
    pl.kernel,
    out_type=jax.ShapeDtypeStruct((B * N * N * C,), jnp.float32),
    mesh=_mesh,
    compiler_params=pltpu.CompilerParams(
        needs_layout_passes=False, use_tc_tiling_on_sc=False
    ),
    scratch_types=[
        pltpu.VMEM((80 * C,), jnp.float32),
        pltpu.VMEM((65 * C,), jnp.float32),
        pltpu.VMEM((65 * C,), jnp.float32),
        pltpu.VMEM((65 * C,), jnp.float32),
        pltpu.VMEM((65 * C,), jnp.float32),
        pltpu.SemaphoreType.DMA,
        pltpu.SemaphoreType.DMA,
        pltpu.SemaphoreType.DMA,
        pltpu.SemaphoreType.DMA,
    ],
)
def _sc_kernel(x_hbm, out_hbm, in_buf, ob0, ob1, ob2, ob3, s0, s1, s2, s3):
    wid = lax.axis_index("s") * 2 + lax.axis_index("c")
    obs = (ob0, ob1, ob2, ob3)
    sems = (s0, s1, s2, s3)
    base_in = wid * (N * C)
    base_out = wid * (N * BLK)

    pltpu.sync_copy(x_hbm.at[pl.ds(base_in, N * C)], in_buf.at[pl.ds(0, N * C)])

    zeros16 = jnp.zeros((16,), jnp.float32)

    def zero_body(j, _):
        for ob in obs:
            for u in range(4):
                ob[pl.ds(j * 64 + u * 16, 16)] = zeros16
        return 0

    lax.fori_loop(0, BLK // 64, zero_body, 0)

    def row_block(i, ob, depth, store_ds):
        def ugroup(g, _):
            gb = g * 64
            for e in (-4, -3, -2, -1):
                j = jnp.maximum(i + e, 0)
                for u in range(4):
                    ob[pl.ds(j * C + gb + 16 * u, 16)] = zeros16
            v = [in_buf[pl.ds(i * C + gb + 16 * u, 16)] for u in range(4)]
            for u in range(4):
                ob[pl.ds(i * C + gb + 16 * u, 16)] = v[u]
            for d in range(1, depth + 1):
                v = [
                    jnp.maximum(v[u], in_buf[pl.ds((i + d) * C + gb + 16 * u, 16)])
                    for u in range(4)
                ]
                if d in store_ds:
                    j = jnp.minimum(i + d, N)
                    for u in range(4):
                        ob[pl.ds(j * C + gb + 16 * u, 16)] = v[u]
            return 0

        lax.fori_loop(0, 4, ugroup, 0)

    def start(i, ob, sem):
        pltpu.async_copy(
            ob.at[pl.ds(0, BLK)], out_hbm.at[pl.ds(base_out + i * BLK, BLK)], sem
        )

    def wait(i, ob, sem):
        pltpu.make_async_copy(
            ob.at[pl.ds(0, BLK)], out_hbm.at[pl.ds(base_out + i * BLK, BLK)], sem
        ).wait()

    def make_quad_body(k):
        i0k = 16 * k

        def body(q, _):
            ibase = i0k + 4 * q
            for r in range(4):
                i = ibase + r
                depth = min(_CLASS_DEPTH[r], 63 - i0k)
                store_ds = sorted(d for d in _O_CLASS[r] if 1 <= d <= depth)
                if k == 0:
                    @pl.when(q > 0)
                    def _():
                        wait(i - 4, obs[r], sems[r])
                else:
                    wait(i - 4, obs[r], sems[r])
                row_block(i, obs[r], depth, store_ds)
                start(i, obs[r], sems[r])
            return 0

        return body

    for k in range(4):
        lax.fori_loop(0, 4, make_quad_body(k), 0)
    for r in range(4):
        wait(60 + r, obs[r], sems[r])


@jax.jit
def kernel(x):
    x_t = jnp.transpose(x, (0, 2, 1)).reshape(B * N * C)
    out = _sc_kernel(x_t)
    x2d = jnp.transpose(out.reshape(B, N, N, C), (0, 3, 1, 2))
    return x2d, jnp.asarray(_MASK2D)


# speedup vs baseline: 4.9006x; 2.1535x over previous
import functools

import jax
import jax.numpy as jnp
import numpy as np
from jax import lax
from jax.experimental import pallas as pl
from jax.experimental.pallas import tpu as pltpu
from jax.experimental.pallas import tpu_sc as plsc

_COUNTS = [15, 8, 8]
B, C, N = 32, 256, 64
NW = 32

_L0 = set(range(15))
_L1 = set(range(16, 31, 2))
_L2 = set(range(34, 63, 4))
_O_CLASS = [_L0 | _L1 | _L2, _L0, _L0 | _L1, _L0]
_CLASS_DEPTH = [62, 14, 30, 14]


def _static_mask() -> np.ndarray:
    mask = np.eye(N, dtype=bool)
    stride, offset = 1, 0
    for _, count in enumerate(_COUNTS):
        for _ in range(count):
            ii = np.arange(0, N - offset, stride)
            mask[ii, ii + offset] = True
            offset += stride
        offset += stride
        stride *= 2
    return mask


_MASK2D = _static_mask()

_mesh = plsc.VectorSubcoreMesh(
    core_axis_name="c", subcore_axis_name="s", num_cores=2, num_subcores=16
)


@functools.partial(
    pl.kernel,
    out_type=jax.ShapeDtypeStruct((B, N, N, C), jnp.float32),
    mesh=_mesh,
    compiler_params=pltpu.CompilerParams(
        needs_layout_passes=False, use_tc_tiling_on_sc=True
    ),
    scratch_types=[
        pltpu.VMEM((80, C), jnp.float32),
        pltpu.VMEM((72, C), jnp.float32),
        pltpu.VMEM((72, C), jnp.float32),
        pltpu.VMEM((72, C), jnp.float32),
        pltpu.VMEM((72, C), jnp.float32),
        pltpu.SemaphoreType.DMA,
        pltpu.SemaphoreType.DMA,
        pltpu.SemaphoreType.DMA,
        pltpu.SemaphoreType.DMA,
    ],
)
def _sc_kernel(x_hbm, out_hbm, in_buf, ob0, ob1, ob2, ob3, s0, s1, s2, s3):
    wid = lax.axis_index("s") * 2 + lax.axis_index("c")
    obs = (ob0, ob1, ob2, ob3)
    sems = (s0, s1, s2, s3)

    pltpu.sync_copy(x_hbm.at[wid], in_buf.at[pl.ds(0, N)])

    zeros16 = jnp.zeros((16,), jnp.float32)

    def zero_body(j, _):
        for ob in obs:
            for u in range(4):
                ob[j // 4, pl.ds((j % 4) * 64 + u * 16, 16)] = zeros16
        return 0

    lax.fori_loop(0, N * 4, zero_body, 0)

    def row_block(i, ob, depth, store_ds):
        def ugroup(g, _):
            gb = g * 64
            for e in (-4, -3, -2, -1):
                j = jnp.maximum(i + e, 0)
                for u in range(4):
                    ob[j, pl.ds(gb + 16 * u, 16)] = zeros16
            v = [in_buf[i, pl.ds(gb + 16 * u, 16)] for u in range(4)]
            for u in range(4):
                ob[i, pl.ds(gb + 16 * u, 16)] = v[u]
            for d in range(1, depth + 1):
                v = [
                    jnp.maximum(v[u], in_buf[i + d, pl.ds(gb + 16 * u, 16)])
                    for u in range(4)
                ]
                if d in store_ds:
                    j = jnp.minimum(i + d, N)
                    for u in range(4):
                        ob[j, pl.ds(gb + 16 * u, 16)] = v[u]
            return 0

        lax.fori_loop(0, 4, ugroup, 0)

    def start(i, ob, sem):
        pltpu.async_copy(ob.at[pl.ds(0, N)], out_hbm.at[wid, i], sem)

    def wait(i, ob, sem):
        pltpu.make_async_copy(ob.at[pl.ds(0, N)], out_hbm.at[wid, i], sem).wait()

    def make_quad_body(k):
        i0k = 16 * k

        def body(q, _):
            ibase = i0k + 4 * q
            for r in range(4):
                i = ibase + r
                depth = min(_CLASS_DEPTH[r], 63 - i0k)
                store_ds = sorted(d for d in _O_CLASS[r] if 1 <= d <= depth)
                if k == 0:
                    @pl.when(q > 0)
                    def _():
                        wait(i - 4, obs[r], sems[r])
                else:
                    wait(i - 4, obs[r], sems[r])
                row_block(i, obs[r], depth, store_ds)
                start(i, obs[r], sems[r])
            return 0

        return body

    for k in range(4):
        lax.fori_loop(0, 4, make_quad_body(k), 0)
    for r in range(4):
        wait(60 + r, obs[r], sems[r])


@jax.jit
def kernel(x):
    x_t = jnp.transpose(x, (0, 2, 1))
    out = _sc_kernel(x_t)
    x2d = jnp.transpose(out, (0, 3, 1, 2))
    return x2d, jnp.asarray(_MASK2D)
